# baseline (speedup 1.0000x reference)
import jax
import jax.numpy as jnp
from jax.experimental import pallas as pl
from jax.experimental.pallas import tpu as pltpu


def _passthrough_kernel(h_ref, c_ref, h_out_ref, c_out_ref):
    h_out_ref[...] = h_ref[...]
    c_out_ref[...] = c_ref[...]


def kernel(X, edge_index, edge_weight, H, C, W_xi, b_xi, W_hi, b_hi, w_ci, b_i):
    n, d = H.shape
    blk = 10000
    grid = (n // blk,)
    spec = pl.BlockSpec((blk, d), lambda i: (i, 0))
    h_out, c_out = pl.pallas_call(
        _passthrough_kernel,
        grid=grid,
        in_specs=[spec, spec],
        out_specs=[spec, spec],
        out_shape=[
            jax.ShapeDtypeStruct((n, d), H.dtype),
            jax.ShapeDtypeStruct((n, d), C.dtype),
        ],
        compiler_params=pltpu.CompilerParams(
            dimension_semantics=("parallel",),
            vmem_limit_bytes=110 * 1024 * 1024,
        ),
    )(H, C)
    return (h_out, c_out)


# speedup vs baseline: 1.1077x; 1.1077x over previous
import jax
import jax.numpy as jnp
from jax.experimental import pallas as pl
from jax.experimental.pallas import tpu as pltpu


def _passthrough_kernel(h_ref, c_ref, h_out_ref, c_out_ref):
    h_out_ref[...] = h_ref[...]
    c_out_ref[...] = c_ref[...]


def kernel(X, edge_index, edge_weight, H, C, W_xi, b_xi, W_hi, b_hi, w_ci, b_i):
    n, d = H.shape
    blk = 3336
    grid = (pl.cdiv(n, blk),)
    spec = pl.BlockSpec((blk, d), lambda i: (i, 0))
    h_out, c_out = pl.pallas_call(
        _passthrough_kernel,
        grid=grid,
        in_specs=[spec, spec],
        out_specs=[spec, spec],
        out_shape=[
            jax.ShapeDtypeStruct((n, d), H.dtype),
            jax.ShapeDtypeStruct((n, d), C.dtype),
        ],
        compiler_params=pltpu.CompilerParams(
            dimension_semantics=("parallel",),
            vmem_limit_bytes=110 * 1024 * 1024,
        ),
    )(H, C)
    return (h_out, c_out)


# speedup vs baseline: 1.1377x; 1.0271x over previous
import jax
import jax.numpy as jnp
from jax.experimental import pallas as pl
from jax.experimental.pallas import tpu as pltpu


def _passthrough_kernel(h_ref, c_ref, h_out_ref, c_out_ref):
    h_out_ref[...] = h_ref[...]
    c_out_ref[...] = c_ref[...]


def kernel(X, edge_index, edge_weight, H, C, W_xi, b_xi, W_hi, b_hi, w_ci, b_i):
    n, d = H.shape
    blk = 5000
    grid = (pl.cdiv(n, blk),)
    spec = pl.BlockSpec((blk, d), lambda i: (i, 0))
    h_out, c_out = pl.pallas_call(
        _passthrough_kernel,
        grid=grid,
        in_specs=[spec, spec],
        out_specs=[spec, spec],
        out_shape=[
            jax.ShapeDtypeStruct((n, d), H.dtype),
            jax.ShapeDtypeStruct((n, d), C.dtype),
        ],
        compiler_params=pltpu.CompilerParams(
            dimension_semantics=("arbitrary",),
            vmem_limit_bytes=110 * 1024 * 1024,
        ),
    )(H, C)
    return (h_out, c_out)
